# Initial kernel scaffold; baseline (speedup 1.0000x reference)
import functools

import jax
import jax.numpy as jnp
import numpy as np
from jax import lax
from jax.experimental import pallas as pl
from jax.experimental.pallas import tpu as pltpu
from jax.experimental.pallas import tpu_sc as plsc

MUL = 32
INV_SQRT_MUL = 1.0 / np.sqrt(32.0)
INV_SQRT_EDGE = 1.0 / np.sqrt(12.0)
SQRT3 = np.float32(np.sqrt(3.0))

NC = 2
NS = 16
NW = NC * NS
CHUNK = 128


def _node_lin_body(nf_ref, w_ref, out_ref):
    out_ref[...] = jnp.dot(nf_ref[...], w_ref[...],
                           preferred_element_type=jnp.float32) * INV_SQRT_MUL


def _edge_prep_body(ee_ref, ev_ref, wfc_ref, out_ref):
    w01 = jnp.dot(ee_ref[...], wfc_ref[...],
                  preferred_element_type=jnp.float32) * INV_SQRT_EDGE
    ev = ev_ref[...]
    nrm = jnp.sqrt(jnp.sum(ev * ev, axis=1, keepdims=True))
    y1 = SQRT3 * ev / (nrm + 1e-12)
    pad = jnp.zeros((ev.shape[0], 5), jnp.float32)
    out_ref[...] = jnp.concatenate([w01, y1, pad], axis=1)


def _post_body(aggp_ref, nf_ref, wq_ref, w1p_ref, r3_ref, out_ref):
    a = aggp_ref[0] + aggp_ref[1]
    s = a[:, :MUL]
    v = a[:, MUL:]
    sn = jnp.concatenate([s, nf_ref[...]], axis=1)
    q = jnp.dot(sn, wq_ref[...], preferred_element_type=jnp.float32) * INV_SQRT_MUL
    g1 = jnp.dot(v, w1p_ref[...], preferred_element_type=jnp.float32) * INV_SQRT_MUL
    scal = q[:, :32] * jax.nn.sigmoid(q[:, :32])
    gates3 = jnp.dot(jax.nn.sigmoid(q[:, 32:48]), r3_ref[...],
                     preferred_element_type=jnp.float32)
    out_ref[...] = jnp.concatenate([scal, g1 * gates3], axis=1)


def _make_sc_kernel(n_nodes, e_pad):
    ew = e_pad // NW
    n_chunks = ew // CHUNK
    rows_per_tile = n_nodes // NS

    mesh = plsc.VectorSubcoreMesh(core_axis_name="c", subcore_axis_name="s")

    @functools.partial(
        pl.kernel,
        out_type=jax.ShapeDtypeStruct((NC, n_nodes, 128), jnp.float32),
        mesh=mesh,
        scratch_types=[
            pltpu.VMEM((CHUNK,), jnp.int32),
            pltpu.VMEM((CHUNK,), jnp.int32),
            pltpu.VMEM((CHUNK, 32), jnp.float32),
            pltpu.VMEM((CHUNK, 72), jnp.float32),
            pltpu.VMEM((CHUNK, 128), jnp.float32),
            pltpu.VMEM_SHARED((n_nodes, 128), jnp.float32),
            pltpu.SemaphoreType.DMA,
        ],
    )
    def sc_kernel(x_hbm, wy_hbm, src_hbm, dst_hbm, out_hbm,
                  dsti, srci, xd, wyc, msg, agg, sem):
        cid = lax.axis_index("c")
        sid = lax.axis_index("s")
        wid = cid * NS + sid

        def zero_msg(i, _):
            msg[i // 8, pl.ds((i % 8) * 16, 16)] = jnp.zeros((16,), jnp.float32)
            return 0
        lax.fori_loop(0, CHUNK * 8, zero_msg, 0)

        base_row = sid * rows_per_tile
        zrows = 125
        def zero_agg(j, _):
            pltpu.sync_copy(msg.at[pl.ds(0, zrows)],
                            agg.at[pl.ds(base_row + j * zrows, zrows)])
            return 0
        lax.fori_loop(0, rows_per_tile // zrows, zero_agg, 0)
        plsc.subcore_barrier()

        ebase = wid * ew

        def chunk_body(ci, _):
            off = ebase + ci * CHUNK
            pltpu.sync_copy(dst_hbm.at[pl.ds(off, CHUNK)], dsti)
            pltpu.sync_copy(src_hbm.at[pl.ds(off, CHUNK)], srci)
            pltpu.sync_copy(wy_hbm.at[pl.ds(off, CHUNK)], wyc)
            pltpu.async_copy(x_hbm.at[dsti], xd, sem).wait()

            def per_edge(e, _):
                xd0 = xd[e, pl.ds(0, 16)]
                xd1 = xd[e, pl.ds(16, 16)]
                msg[e, pl.ds(0, 16)] = xd0 * wyc[e, pl.ds(0, 16)]
                msg[e, pl.ds(16, 16)] = xd1 * wyc[e, pl.ds(16, 16)]
                h0 = xd0 * wyc[e, pl.ds(32, 16)]
                h1 = xd1 * wyc[e, pl.ds(48, 16)]
                yx = wyc[e, 64]
                yy = wyc[e, 65]
                yz = wyc[e, 66]
                msg[e, pl.ds(32, 16)] = h0 * yx
                msg[e, pl.ds(48, 16)] = h1 * yx
                msg[e, pl.ds(64, 16)] = h0 * yy
                msg[e, pl.ds(80, 16)] = h1 * yy
                msg[e, pl.ds(96, 16)] = h0 * yz
                msg[e, pl.ds(112, 16)] = h1 * yz
                return 0
            lax.fori_loop(0, CHUNK, per_edge, 0)

            pltpu.sync_copy(msg, agg.at[srci], add=True)
            return 0
        lax.fori_loop(0, n_chunks, chunk_body, 0)
        plsc.subcore_barrier()

        pltpu.sync_copy(agg.at[pl.ds(base_row, rows_per_tile)],
                        out_hbm.at[cid, pl.ds(base_row, rows_per_tile)])

    return sc_kernel


def kernel(node_feature, edge_index, edge_vec, edge_embedding,
           W_fc, W_nl0, W_skip0, W_nl2_0, W_nl2_1):
    n = node_feature.shape[0]
    e = edge_index.shape[1]

    e_pad = ((e + NW * CHUNK - 1) // (NW * CHUNK)) * (NW * CHUNK)
    pad = e_pad - e
    src = jnp.pad(edge_index[0], (0, pad))
    dst = jnp.pad(edge_index[1], (0, pad))
    eep = jnp.pad(edge_embedding, ((0, pad), (0, 0)))
    evp = jnp.pad(edge_vec, ((0, pad), (0, 0)))

    x = pl.pallas_call(
        _node_lin_body,
        out_shape=jax.ShapeDtypeStruct((n, MUL), jnp.float32),
    )(node_feature, W_nl0)

    BE = 2048
    wy = pl.pallas_call(
        _edge_prep_body,
        grid=(e_pad // BE,),
        in_specs=[
            pl.BlockSpec((BE, 12), lambda i: (i, 0)),
            pl.BlockSpec((BE, 3), lambda i: (i, 0)),
            pl.BlockSpec((12, 64), lambda i: (0, 0)),
        ],
        out_specs=pl.BlockSpec((BE, 72), lambda i: (i, 0)),
        out_shape=jax.ShapeDtypeStruct((e_pad, 72), jnp.float32),
    )(eep, evp, W_fc[:, :64])

    aggp = _make_sc_kernel(n, e_pad)(x, wy, src, dst)

    Wq = jnp.concatenate([W_nl2_0, W_skip0], axis=0)
    W1p = jnp.zeros((96, 48), jnp.float32)
    for c in range(3):
        W1p = W1p.at[32 * c:32 * (c + 1), c::3].set(W_nl2_1)
    R3 = jnp.asarray(np.kron(np.eye(16, dtype=np.float32),
                             np.ones((1, 3), np.float32)))

    BN = 2000
    out = pl.pallas_call(
        _post_body,
        grid=(n // BN,),
        in_specs=[
            pl.BlockSpec((NC, BN, 128), lambda i: (0, i, 0)),
            pl.BlockSpec((BN, MUL), lambda i: (i, 0)),
            pl.BlockSpec((64, 48), lambda i: (0, 0)),
            pl.BlockSpec((96, 48), lambda i: (0, 0)),
            pl.BlockSpec((16, 48), lambda i: (0, 0)),
        ],
        out_specs=pl.BlockSpec((BN, 80), lambda i: (i, 0)),
        out_shape=jax.ShapeDtypeStruct((n, 80), jnp.float32),
    )(aggp, node_feature, Wq, W1p, R3)
    return out


# speedup vs baseline: 2.8958x; 2.8958x over previous
import functools

import jax
import jax.numpy as jnp
import numpy as np
from jax import lax
from jax.experimental import pallas as pl
from jax.experimental.pallas import tpu as pltpu
from jax.experimental.pallas import tpu_sc as plsc

MUL = 32
INV_SQRT_MUL = 1.0 / np.sqrt(32.0)
INV_SQRT_EDGE = 1.0 / np.sqrt(12.0)
SQRT3 = np.float32(np.sqrt(3.0))

NC = 2
NS = 16
NW = NC * NS
CHUNK = 128


def _node_lin_body(nf_ref, w_ref, out_ref):
    x = jnp.dot(nf_ref[...], w_ref[...],
                preferred_element_type=jnp.float32) * INV_SQRT_MUL
    out_ref[...] = jnp.concatenate(
        [x, jnp.zeros((x.shape[0], 96), jnp.float32)], axis=1)


def _edge_prep_body(ee_ref, ev_ref, wfc_ref, out_ref):
    w01 = jnp.dot(ee_ref[...], wfc_ref[...],
                  preferred_element_type=jnp.float32) * INV_SQRT_EDGE
    ev = ev_ref[...]
    nrm = jnp.sqrt(jnp.sum(ev * ev, axis=1, keepdims=True))
    y1 = SQRT3 * ev / (nrm + 1e-12)
    pad = jnp.zeros((ev.shape[0], 5), jnp.float32)
    out_ref[...] = jnp.concatenate([w01, y1, pad], axis=1)


def _post_body(aggp_ref, nf_ref, wq_ref, w1p_ref, r3_ref, out_ref):
    a = aggp_ref[0] + aggp_ref[1]
    s = a[:, :MUL]
    v = a[:, MUL:]
    sn = jnp.concatenate([s, nf_ref[...]], axis=1)
    q = jnp.dot(sn, wq_ref[...], preferred_element_type=jnp.float32) * INV_SQRT_MUL
    g1 = jnp.dot(v, w1p_ref[...], preferred_element_type=jnp.float32) * INV_SQRT_MUL
    scal = q[:, :32] * jax.nn.sigmoid(q[:, :32])
    gates3 = jnp.dot(jax.nn.sigmoid(q[:, 32:48]), r3_ref[...],
                     preferred_element_type=jnp.float32)
    out_ref[...] = jnp.concatenate([scal, g1 * gates3], axis=1)


def _make_sc_kernel(n_pad, e_pad):
    ew = e_pad // NW
    n_chunks = ew // CHUNK
    rows_per_tile = n_pad // NS

    mesh = plsc.VectorSubcoreMesh(core_axis_name="c", subcore_axis_name="s")

    @functools.partial(
        pl.kernel,
        out_type=jax.ShapeDtypeStruct((NC, n_pad, 128), jnp.float32),
        mesh=mesh,
        scratch_types=[
            pltpu.VMEM((CHUNK,), jnp.int32),
            pltpu.VMEM((CHUNK,), jnp.int32),
            pltpu.VMEM((CHUNK, 128), jnp.float32),
            pltpu.VMEM((CHUNK, 72), jnp.float32),
            pltpu.VMEM((CHUNK, 128), jnp.float32),
            pltpu.VMEM_SHARED((n_pad, 128), jnp.float32),
            pltpu.SemaphoreType.DMA,
        ],
    )
    def sc_kernel(x_hbm, wy_hbm, src_hbm, dst_hbm, out_hbm,
                  dsti, srci, xd, wyc, msg, agg, sem):
        cid = lax.axis_index("c")
        sid = lax.axis_index("s")
        wid = cid * NS + sid

        def zero_msg(i, _):
            msg[i // 8, pl.ds((i % 8) * 16, 16)] = jnp.zeros((16,), jnp.float32)
            return 0
        lax.fori_loop(0, CHUNK * 8, zero_msg, 0)

        base_row = sid * rows_per_tile
        def zero_agg(j, _):
            pltpu.sync_copy(msg,
                            agg.at[pl.ds(base_row + j * CHUNK, CHUNK)])
            return 0
        lax.fori_loop(0, rows_per_tile // CHUNK, zero_agg, 0)
        rem = rows_per_tile % CHUNK
        if rem:
            pltpu.sync_copy(
                msg.at[pl.ds(0, rem)],
                agg.at[pl.ds(base_row + (rows_per_tile - rem), rem)])
        plsc.subcore_barrier()

        ebase = wid * ew

        def chunk_body(ci, _):
            off = ebase + ci * CHUNK
            pltpu.sync_copy(dst_hbm.at[pl.ds(off, CHUNK)], dsti)
            pltpu.sync_copy(src_hbm.at[pl.ds(off, CHUNK)], srci)
            pltpu.sync_copy(wy_hbm.at[pl.ds(off, CHUNK)], wyc)
            pltpu.async_copy(x_hbm.at[dsti], xd, sem).wait()

            def per_edge(e, _):
                xd0 = xd[e, pl.ds(0, 16)]
                xd1 = xd[e, pl.ds(16, 16)]
                msg[e, pl.ds(0, 16)] = xd0 * wyc[e, pl.ds(0, 16)]
                msg[e, pl.ds(16, 16)] = xd1 * wyc[e, pl.ds(16, 16)]
                h0 = xd0 * wyc[e, pl.ds(32, 16)]
                h1 = xd1 * wyc[e, pl.ds(48, 16)]
                yv = wyc[e, pl.ds(56, 16)]
                yx = yv[8]
                yy = yv[9]
                yz = yv[10]
                msg[e, pl.ds(32, 16)] = h0 * yx
                msg[e, pl.ds(48, 16)] = h1 * yx
                msg[e, pl.ds(64, 16)] = h0 * yy
                msg[e, pl.ds(80, 16)] = h1 * yy
                msg[e, pl.ds(96, 16)] = h0 * yz
                msg[e, pl.ds(112, 16)] = h1 * yz
                return 0
            lax.fori_loop(0, CHUNK, per_edge, 0)

            pltpu.sync_copy(msg, agg.at[srci], add=True)
            return 0
        lax.fori_loop(0, n_chunks, chunk_body, 0)
        plsc.subcore_barrier()

        pltpu.sync_copy(agg.at[pl.ds(base_row, rows_per_tile)],
                        out_hbm.at[cid, pl.ds(base_row, rows_per_tile)])

    return sc_kernel


def kernel(node_feature, edge_index, edge_vec, edge_embedding,
           W_fc, W_nl0, W_skip0, W_nl2_0, W_nl2_1):
    n = node_feature.shape[0]
    e = edge_index.shape[1]

    e_pad = ((e + NW * CHUNK - 1) // (NW * CHUNK)) * (NW * CHUNK)
    pad = e_pad - e
    src = jnp.pad(edge_index[0], (0, pad))
    dst = jnp.pad(edge_index[1], (0, pad))
    eep = jnp.pad(edge_embedding, ((0, pad), (0, 0)))
    evp = jnp.pad(edge_vec, ((0, pad), (0, 0)))

    x = pl.pallas_call(
        _node_lin_body,
        out_shape=jax.ShapeDtypeStruct((n, 128), jnp.float32),
    )(node_feature, W_nl0)

    BE = 2048
    wy = pl.pallas_call(
        _edge_prep_body,
        grid=(e_pad // BE,),
        in_specs=[
            pl.BlockSpec((BE, 12), lambda i: (i, 0)),
            pl.BlockSpec((BE, 3), lambda i: (i, 0)),
            pl.BlockSpec((12, 64), lambda i: (0, 0)),
        ],
        out_specs=pl.BlockSpec((BE, 72), lambda i: (i, 0)),
        out_shape=jax.ShapeDtypeStruct((e_pad, 72), jnp.float32),
    )(eep, evp, W_fc[:, :64])

    n_pad = ((n + NS * 8 - 1) // (NS * 8)) * (NS * 8)
    aggp = _make_sc_kernel(n_pad, e_pad)(x, wy, src, dst)

    Wq = jnp.concatenate([W_nl2_0, W_skip0], axis=0)
    W1p = jnp.zeros((96, 48), jnp.float32)
    for c in range(3):
        W1p = W1p.at[32 * c:32 * (c + 1), c::3].set(W_nl2_1)
    R3 = jnp.asarray(np.kron(np.eye(16, dtype=np.float32),
                             np.ones((1, 3), np.float32)))

    BN = 2000
    out = pl.pallas_call(
        _post_body,
        grid=(n // BN,),
        in_specs=[
            pl.BlockSpec((NC, BN, 128), lambda i: (0, i, 0)),

            pl.BlockSpec((BN, MUL), lambda i: (i, 0)),
            pl.BlockSpec((64, 48), lambda i: (0, 0)),
            pl.BlockSpec((96, 48), lambda i: (0, 0)),
            pl.BlockSpec((16, 48), lambda i: (0, 0)),
        ],
        out_specs=pl.BlockSpec((BN, 80), lambda i: (i, 0)),
        out_shape=jax.ShapeDtypeStruct((n, 80), jnp.float32),
    )(aggp, node_feature, Wq, W1p, R3)
    return out


# speedup vs baseline: 3.4758x; 1.2003x over previous
import functools

import jax
import jax.numpy as jnp
import numpy as np
from jax import lax
from jax.experimental import pallas as pl
from jax.experimental.pallas import tpu as pltpu
from jax.experimental.pallas import tpu_sc as plsc

MUL = 32
INV_SQRT_MUL = 1.0 / np.sqrt(32.0)
INV_SQRT_EDGE = 1.0 / np.sqrt(12.0)
SQRT3 = np.float32(np.sqrt(3.0))

NC = 2
NS = 16
NW = NC * NS
CHUNK = 48


def _node_lin_body(nf_ref, w_ref, out_ref):
    x = jnp.dot(nf_ref[...], w_ref[...],
                preferred_element_type=jnp.float32) * INV_SQRT_MUL
    out_ref[...] = jnp.concatenate(
        [x, jnp.zeros((x.shape[0], 96), jnp.float32)], axis=1)


def _edge_prep_body(ee_ref, ev_ref, wfc_ref, out_ref):
    w01 = jnp.dot(ee_ref[...], wfc_ref[...],
                  preferred_element_type=jnp.float32) * INV_SQRT_EDGE
    ev = ev_ref[...]
    nrm = jnp.sqrt(jnp.sum(ev * ev, axis=1, keepdims=True))
    y1 = SQRT3 * ev / (nrm + 1e-12)
    pad = jnp.zeros((ev.shape[0], 5), jnp.float32)
    out_ref[...] = jnp.concatenate([w01, y1, pad], axis=1)


def _post_body(aggp_ref, nf_ref, wq_ref, w1p_ref, r3_ref, out_ref):
    a = aggp_ref[0] + aggp_ref[1]
    s = a[:, :MUL]
    v = a[:, MUL:]
    sn = jnp.concatenate([s, nf_ref[...]], axis=1)
    q = jnp.dot(sn, wq_ref[...], preferred_element_type=jnp.float32) * INV_SQRT_MUL
    g1 = jnp.dot(v, w1p_ref[...], preferred_element_type=jnp.float32) * INV_SQRT_MUL
    scal = q[:, :32] * jax.nn.sigmoid(q[:, :32])
    gates3 = jnp.dot(jax.nn.sigmoid(q[:, 32:48]), r3_ref[...],
                     preferred_element_type=jnp.float32)
    out_ref[...] = jnp.concatenate([scal, g1 * gates3], axis=1)


def _make_sc_kernel(n_pad, e_pad):
    ew = e_pad // NW
    n_chunks = ew // CHUNK
    rows_per_tile = n_pad // NS

    assert n_chunks % 6 == 0
    mesh = plsc.VectorSubcoreMesh(core_axis_name="c", subcore_axis_name="s")

    @functools.partial(
        pl.kernel,
        out_type=jax.ShapeDtypeStruct((NC, n_pad, 128), jnp.float32),
        mesh=mesh,
        scratch_types=[
            pltpu.VMEM((2, CHUNK), jnp.int32),
            pltpu.VMEM((3, CHUNK), jnp.int32),
            pltpu.VMEM((2, CHUNK, 128), jnp.float32),
            pltpu.VMEM((2, CHUNK, 72), jnp.float32),
            pltpu.VMEM((3, CHUNK, 128), jnp.float32),
            pltpu.VMEM_SHARED((n_pad, 128), jnp.float32),
            pltpu.SemaphoreType.DMA,
            pltpu.SemaphoreType.DMA,
            pltpu.SemaphoreType.DMA,
            pltpu.SemaphoreType.DMA,
            pltpu.SemaphoreType.DMA,
            pltpu.SemaphoreType.DMA,
            pltpu.SemaphoreType.DMA,
        ],
    )
    def sc_kernel(x_hbm, wy_hbm, src_hbm, dst_hbm, out_hbm,
                  dsti, srci, xd, wyc, msg, agg,
                  sg0, sg1, sw0, sw1, ss0, ss1, ss2):
        sgs = (sg0, sg1)
        sws = (sw0, sw1)
        sss = (ss0, ss1, ss2)
        cid = lax.axis_index("c")
        sid = lax.axis_index("s")
        wid = cid * NS + sid

        zb = msg.at[0]
        def zero_msg(i, _):
            zb[i // 8, pl.ds((i % 8) * 16, 16)] = jnp.zeros((16,), jnp.float32)
            return 0
        lax.fori_loop(0, CHUNK * 8, zero_msg, 0)

        base_row = sid * rows_per_tile
        def zero_agg(j, _):
            pltpu.sync_copy(zb, agg.at[pl.ds(base_row + j * CHUNK, CHUNK)])
            return 0
        lax.fori_loop(0, rows_per_tile // CHUNK, zero_agg, 0)
        rem = rows_per_tile % CHUNK
        if rem:
            pltpu.sync_copy(
                zb.at[pl.ds(0, rem)],
                agg.at[pl.ds(base_row + (rows_per_tile - rem), rem)])
        plsc.subcore_barrier()

        ebase = wid * ew

        def load_chunk(ci, s2, s3):
            off = ebase + ci * CHUNK
            pltpu.sync_copy(dst_hbm.at[pl.ds(off, CHUNK)], dsti.at[s2])
            pltpu.sync_copy(src_hbm.at[pl.ds(off, CHUNK)], srci.at[s3])
            pltpu.async_copy(x_hbm.at[dsti.at[s2]], xd.at[s2], sgs[s2])
            pltpu.async_copy(wy_hbm.at[pl.ds(off, CHUNK)], wyc.at[s2], sws[s2])

        load_chunk(0, 0, 0)

        def group(it, _):
            for j in range(6):
                ci = it * 6 + j
                s2 = j % 2
                s3 = j % 3
                n2 = (j + 1) % 2
                n3 = (j + 1) % 3

                @pl.when(ci >= 2)
                def _():
                    pltpu.make_async_copy(
                        msg.at[n3], agg.at[srci.at[n3]], sss[n3]).wait()

                @pl.when(ci + 1 < n_chunks)
                def _():
                    load_chunk(ci + 1, n2, n3)

                pltpu.make_async_copy(
                    x_hbm.at[dsti.at[s2]], xd.at[s2], sgs[s2]).wait()
                pltpu.make_async_copy(
                    wy_hbm.at[pl.ds(0, CHUNK)], wyc.at[s2], sws[s2]).wait()

                msg_s = msg.at[s3]
                xd_s = xd.at[s2]
                wyc_s = wyc.at[s2]

                def per_edge(e, _):
                    xd0 = xd_s[e, pl.ds(0, 16)]
                    xd1 = xd_s[e, pl.ds(16, 16)]
                    msg_s[e, pl.ds(0, 16)] = xd0 * wyc_s[e, pl.ds(0, 16)]
                    msg_s[e, pl.ds(16, 16)] = xd1 * wyc_s[e, pl.ds(16, 16)]
                    h0 = xd0 * wyc_s[e, pl.ds(32, 16)]
                    h1 = xd1 * wyc_s[e, pl.ds(48, 16)]
                    yv = wyc_s[e, pl.ds(56, 16)]
                    yx = yv[8]
                    yy = yv[9]
                    yz = yv[10]
                    msg_s[e, pl.ds(32, 16)] = h0 * yx
                    msg_s[e, pl.ds(48, 16)] = h1 * yx
                    msg_s[e, pl.ds(64, 16)] = h0 * yy
                    msg_s[e, pl.ds(80, 16)] = h1 * yy
                    msg_s[e, pl.ds(96, 16)] = h0 * yz
                    msg_s[e, pl.ds(112, 16)] = h1 * yz
                    return 0
                lax.fori_loop(0, CHUNK, per_edge, 0)

                pltpu.async_copy(msg.at[s3], agg.at[srci.at[s3]],
                                 sss[s3], add=True)
            return 0
        lax.fori_loop(0, n_chunks // 6, group, 0)

        for s in ((n_chunks - 2) % 3, (n_chunks - 1) % 3):
            pltpu.make_async_copy(
                msg.at[s], agg.at[srci.at[s]], sss[s]).wait()
        plsc.subcore_barrier()

        pltpu.sync_copy(agg.at[pl.ds(base_row, rows_per_tile)],
                        out_hbm.at[cid, pl.ds(base_row, rows_per_tile)])

    return sc_kernel


def kernel(node_feature, edge_index, edge_vec, edge_embedding,
           W_fc, W_nl0, W_skip0, W_nl2_0, W_nl2_1):
    n = node_feature.shape[0]
    e = edge_index.shape[1]

    quantum = NW * CHUNK * 6
    e_pad = ((e + quantum - 1) // quantum) * quantum
    pad = e_pad - e
    src = jnp.pad(edge_index[0], (0, pad))
    dst = jnp.pad(edge_index[1], (0, pad))
    eep = jnp.pad(edge_embedding, ((0, pad), (0, 0)))
    evp = jnp.pad(edge_vec, ((0, pad), (0, 0)))

    x = pl.pallas_call(
        _node_lin_body,
        out_shape=jax.ShapeDtypeStruct((n, 128), jnp.float32),
    )(node_feature, W_nl0)

    BE = 2048
    wy = pl.pallas_call(
        _edge_prep_body,
        grid=(e_pad // BE,),
        in_specs=[
            pl.BlockSpec((BE, 12), lambda i: (i, 0)),
            pl.BlockSpec((BE, 3), lambda i: (i, 0)),
            pl.BlockSpec((12, 64), lambda i: (0, 0)),
        ],
        out_specs=pl.BlockSpec((BE, 72), lambda i: (i, 0)),
        out_shape=jax.ShapeDtypeStruct((e_pad, 72), jnp.float32),
    )(eep, evp, W_fc[:, :64])

    n_pad = ((n + NS * 8 - 1) // (NS * 8)) * (NS * 8)
    aggp = _make_sc_kernel(n_pad, e_pad)(x, wy, src, dst)

    Wq = jnp.concatenate([W_nl2_0, W_skip0], axis=0)
    W1p = jnp.zeros((96, 48), jnp.float32)
    for c in range(3):
        W1p = W1p.at[32 * c:32 * (c + 1), c::3].set(W_nl2_1)
    R3 = jnp.asarray(np.kron(np.eye(16, dtype=np.float32),
                             np.ones((1, 3), np.float32)))

    BN = 2000
    out = pl.pallas_call(
        _post_body,
        grid=(n // BN,),
        in_specs=[
            pl.BlockSpec((NC, BN, 128), lambda i: (0, i, 0)),

            pl.BlockSpec((BN, MUL), lambda i: (i, 0)),
            pl.BlockSpec((64, 48), lambda i: (0, 0)),
            pl.BlockSpec((96, 48), lambda i: (0, 0)),
            pl.BlockSpec((16, 48), lambda i: (0, 0)),
        ],
        out_specs=pl.BlockSpec((BN, 80), lambda i: (i, 0)),
        out_shape=jax.ShapeDtypeStruct((n, 80), jnp.float32),
    )(aggp, node_feature, Wq, W1p, R3)
    return out
